# Initial kernel scaffold; baseline (speedup 1.0000x reference)
import jax
import jax.numpy as jnp
from jax.experimental import pallas as pl


def kernel(nfeats, efeats, edge_index, W_node, b_node, W_ni, W_nj, W_fij, bias_e):
    raise NotImplementedError("write your pallas kernel here")


# speedup vs baseline: 2.8962x; 2.8962x over previous
import functools

import jax
import jax.numpy as jnp
from jax import lax
from jax.experimental import pallas as pl
from jax.experimental.pallas import tpu as pltpu
from jax.experimental.pallas import tpu_sc as plsc

N_NODES = 10000
N_EDGES = 320000
D_IN = 128
D_H = D_IN // 2
D_E = 16

NC = 2
NS = 16
EPW = N_EDGES // NS
CH = 80
NCH = EPW // CH
N_PAD = 10240
RPT = N_PAD // NS
ZR = 128

_SC_MESH = plsc.VectorSubcoreMesh(core_axis_name="c", subcore_axis_name="s")


def _sc_hagg_body(src_hbm, dst_hbm, nfcols_hbm, s_out, d_out,
                  srcv, srcv2, dstv, rows, ones_b, zbuf, zd, s_sh, d_sh, gsem):
    cid = lax.axis_index("c")
    sid = lax.axis_index("s")

    z16 = jnp.zeros((16,), jnp.float32)
    one16 = jnp.ones((16,), jnp.float32)

    def _zrow(i, _):
        for c in range(D_H // 16):
            zbuf[i, pl.ds(c * 16, 16)] = z16
        return _
    lax.fori_loop(0, ZR, _zrow, None)

    def _zdeg(i, _):
        zd[i] = z16
        return _
    lax.fori_loop(0, RPT, _zdeg, None)

    def _ones(i, _):
        ones_b[i] = one16
        return _
    lax.fori_loop(0, CH, _ones, None)

    for k in range(RPT // ZR):
        pltpu.sync_copy(zbuf, s_sh.at[pl.ds(sid * RPT + k * ZR, ZR)])
    pltpu.sync_copy(zd, d_sh.at[pl.ds(sid * RPT, RPT)])
    plsc.subcore_barrier()

    base = sid * EPW
    row_off = cid * N_NODES

    def _chunk(j, _):
        off = base + j * CH
        pltpu.sync_copy(src_hbm.at[pl.ds(off, CH)], srcv)
        pltpu.sync_copy(dst_hbm.at[pl.ds(off, CH)], dstv)
        for k in range(CH // 16):
            sl = pl.ds(k * 16, 16)
            srcv2[sl] = srcv[sl] + row_off
        pltpu.async_copy(nfcols_hbm.at[srcv2], rows, gsem).wait()
        pltpu.sync_copy(rows, s_sh.at[dstv], add=True)

        @pl.when((j < NCH // 2) == (cid == 0))
        def _():
            pltpu.sync_copy(ones_b, d_sh.at[dstv], add=True)
        return _
    lax.fori_loop(0, NCH, _chunk, None)

    plsc.subcore_barrier()

    rs = sid * RPT
    pltpu.sync_copy(s_sh.at[pl.ds(rs, RPT)],
                    s_out.at[pl.ds(cid * N_PAD + rs, RPT)])
    pltpu.sync_copy(d_sh.at[pl.ds(rs, RPT)],
                    d_out.at[pl.ds(cid * N_PAD + rs, RPT)])


_sc_hagg = pl.kernel(
    _sc_hagg_body,
    out_type=[jax.ShapeDtypeStruct((2 * N_PAD, D_H), jnp.float32),
              jax.ShapeDtypeStruct((2 * N_PAD, 16), jnp.float32)],
    mesh=_SC_MESH,
    compiler_params=pltpu.CompilerParams(use_tc_tiling_on_sc=False),
    scratch_types=[
        pltpu.VMEM((CH,), jnp.int32),
        pltpu.VMEM((CH,), jnp.int32),
        pltpu.VMEM((CH,), jnp.int32),
        pltpu.VMEM((CH, D_H), jnp.float32),
        pltpu.VMEM((CH, 16), jnp.float32),
        pltpu.VMEM((ZR, D_H), jnp.float32),
        pltpu.VMEM((RPT, 16), jnp.float32),
        pltpu.VMEM_SHARED((N_PAD, D_H), jnp.float32),
        pltpu.VMEM_SHARED((N_PAD, 16), jnp.float32),
        pltpu.SemaphoreType.DMA,
    ],
)


NW = NC * NS
EPW_F = N_EDGES // NW
NCH_F = EPW_F // CH


def _sc_fasm_body(src_hbm, dst_hbm, fni_hbm, fnj_hbm, ein_hbm, f_out,
                  srcv, dstv, abuf, bbuf, cbuf, obuf, asem, bsem):
    cid = lax.axis_index("c")
    sid = lax.axis_index("s")
    wid = sid * NC + cid
    base = wid * EPW_F

    def _chunk(j, _):
        off = base + j * CH
        pltpu.sync_copy(src_hbm.at[pl.ds(off, CH)], srcv)
        pltpu.sync_copy(dst_hbm.at[pl.ds(off, CH)], dstv)
        cpa = pltpu.async_copy(fni_hbm.at[srcv], abuf, asem)
        cpb = pltpu.async_copy(fnj_hbm.at[dstv], bbuf, bsem)
        pltpu.sync_copy(ein_hbm.at[pl.ds(off, CH)], cbuf)
        cpa.wait()
        cpb.wait()

        def _edge(i, _):
            x = abuf[i] + bbuf[i] + cbuf[i]
            obuf[i] = jnp.maximum(x, x * 0.01)
            return _
        lax.fori_loop(0, CH, _edge, None)

        pltpu.sync_copy(obuf, f_out.at[pl.ds(off, CH)])
        return _
    lax.fori_loop(0, NCH_F, _chunk, None)


_sc_fasm = pl.kernel(
    _sc_fasm_body,
    out_type=jax.ShapeDtypeStruct((N_EDGES, D_E), jnp.float32),
    mesh=_SC_MESH,
    compiler_params=pltpu.CompilerParams(use_tc_tiling_on_sc=False),
    scratch_types=[
        pltpu.VMEM((CH,), jnp.int32),
        pltpu.VMEM((CH,), jnp.int32),
        pltpu.VMEM((CH, D_E), jnp.float32),
        pltpu.VMEM((CH, D_E), jnp.float32),
        pltpu.VMEM((CH, D_E), jnp.float32),
        pltpu.VMEM((CH, D_E), jnp.float32),
        pltpu.SemaphoreType.DMA,
        pltpu.SemaphoreType.DMA,
    ],
)


def _tc_node_tables_body(x_ref, wni_ref, wnj_ref, oni_ref, onj_ref):
    x = x_ref[...]
    dn = (((1,), (1,)), ((), ()))
    oni_ref[...] = lax.dot_general(x, wni_ref[...], dn,
                                   preferred_element_type=jnp.float32)
    onj_ref[...] = lax.dot_general(x, wnj_ref[...], dn,
                                   preferred_element_type=jnp.float32)


def _tc_node_tables(nfeats, W_ni, W_nj):
    blk = 1000
    return pl.pallas_call(
        _tc_node_tables_body,
        grid=(N_NODES // blk,),
        in_specs=[pl.BlockSpec((blk, D_IN), lambda i: (i, 0)),
                  pl.BlockSpec((D_E, D_IN), lambda i: (0, 0)),
                  pl.BlockSpec((D_E, D_IN), lambda i: (0, 0))],
        out_specs=[pl.BlockSpec((blk, D_E), lambda i: (i, 0)),
                   pl.BlockSpec((blk, D_E), lambda i: (i, 0))],
        out_shape=[jax.ShapeDtypeStruct((N_NODES, D_E), jnp.float32),
                   jax.ShapeDtypeStruct((N_NODES, D_E), jnp.float32)],
    )(nfeats, W_ni, W_nj)


def _tc_edge_proj_body(e_ref, w_ref, b_ref, o_ref):
    dn = (((1,), (1,)), ((), ()))
    o_ref[...] = lax.dot_general(e_ref[...], w_ref[...], dn,
                                 preferred_element_type=jnp.float32) + b_ref[...]


def _tc_edge_proj(efeats, W_fij, bias_e):
    blk = 2000
    return pl.pallas_call(
        _tc_edge_proj_body,
        grid=(N_EDGES // blk,),
        in_specs=[pl.BlockSpec((blk, D_E), lambda i: (i, 0)),
                  pl.BlockSpec((D_E, D_E), lambda i: (0, 0)),
                  pl.BlockSpec((1, D_E), lambda i: (0, 0))],
        out_specs=pl.BlockSpec((blk, D_E), lambda i: (i, 0)),
        out_shape=jax.ShapeDtypeStruct((N_EDGES, D_E), jnp.float32),
    )(efeats, W_fij, bias_e)


def _tc_node_out_body(s_ref, d_ref, w_ref, b_ref, o_ref):
    s = jnp.concatenate([s_ref[0], s_ref[1]], axis=1)
    deg = (d_ref[0] + d_ref[1])[:, 0:1]
    acc = lax.dot_general(s, w_ref[...], (((1,), (1,)), ((), ())),
                          preferred_element_type=jnp.float32)
    o_ref[...] = acc / jnp.maximum(deg, 1.0) + jnp.where(deg > 0.0,
                                                         b_ref[...], 0.0)


def _tc_node_out(s2, d2, W_node, b_node):
    blk = 1000
    return pl.pallas_call(
        _tc_node_out_body,
        grid=(N_NODES // blk,),
        in_specs=[pl.BlockSpec((2, blk, D_H), lambda i: (0, i, 0)),
                  pl.BlockSpec((2, blk, 16), lambda i: (0, i, 0)),
                  pl.BlockSpec((D_IN, D_IN), lambda i: (0, 0)),
                  pl.BlockSpec((1, D_IN), lambda i: (0, 0))],
        out_specs=pl.BlockSpec((blk, D_IN), lambda i: (i, 0)),
        out_shape=jax.ShapeDtypeStruct((N_NODES, D_IN), jnp.float32),
    )(s2, d2, W_node, b_node)


def kernel(nfeats, efeats, edge_index, W_node, b_node, W_ni, W_nj, W_fij, bias_e):
    src = edge_index[0]
    dst = edge_index[1]
    nf_cols = jnp.concatenate([nfeats[:, :D_H], nfeats[:, D_H:]], axis=0)

    s2, d2 = _sc_hagg(src, dst, nf_cols)
    f_ni, f_nj = _tc_node_tables(nfeats, W_ni, W_nj)
    e_in = _tc_edge_proj(efeats, W_fij, bias_e.reshape(1, D_E))
    f_out = _sc_fasm(src, dst, f_ni, f_nj, e_in)
    h_out = _tc_node_out(s2.reshape(2, N_PAD, D_H),
                         d2.reshape(2, N_PAD, 16),
                         W_node, b_node.reshape(1, D_IN))
    return h_out, f_out


# speedup vs baseline: 4.5269x; 1.5630x over previous
import functools

import jax
import jax.numpy as jnp
from jax import lax
from jax.experimental import pallas as pl
from jax.experimental.pallas import tpu as pltpu
from jax.experimental.pallas import tpu_sc as plsc

N_NODES = 10000
N_EDGES = 320000
D_IN = 128
D_H = D_IN // 2
D_E = 16

NC = 2
NS = 16
CH = 80
IDX_ROWS = N_EDGES // CH

EPW = N_EDGES // NS
NCH = EPW // CH
N_PAD = 10240
RPT = N_PAD // NS
ZR = 128

_SC_MESH = plsc.VectorSubcoreMesh(core_axis_name="c", subcore_axis_name="s")


def _sc_hagg_body(src2_hbm, dst2_hbm, nflo_hbm, nfhi_hbm, s_out, d_out,
                  src_all, dst_all, rows, ones_b, zbuf, zd, s_sh, d_sh, gsem):
    cid = lax.axis_index("c")
    sid = lax.axis_index("s")

    z16 = jnp.zeros((16,), jnp.float32)
    one16 = jnp.ones((16,), jnp.float32)

    def _zrow(i, _):
        for c in range(D_H // 16):
            zbuf[i, pl.ds(c * 16, 16)] = z16
        return _
    lax.fori_loop(0, ZR, _zrow, None)

    def _zdeg(i, _):
        zd[i] = z16
        return _
    lax.fori_loop(0, RPT, _zdeg, None)

    def _ones(i, _):
        ones_b[i] = one16
        return _
    lax.fori_loop(0, CH, _ones, None)

    pltpu.sync_copy(src2_hbm.at[pl.ds(sid * NCH, NCH)], src_all)
    pltpu.sync_copy(dst2_hbm.at[pl.ds(sid * NCH, NCH)], dst_all)

    for k in range(RPT // ZR):
        pltpu.sync_copy(zbuf, s_sh.at[pl.ds(sid * RPT + k * ZR, ZR)])
    pltpu.sync_copy(zd, d_sh.at[pl.ds(sid * RPT, RPT)])
    plsc.subcore_barrier()

    def _run(tbl, count_first_half):
        pltpu.async_copy(tbl.at[src_all.at[0]], rows.at[0], gsem)

        def _iter(j, _):
            b = lax.rem(j, 2)
            @pl.when(j + 1 < NCH)
            def _():
                pltpu.async_copy(tbl.at[src_all.at[j + 1]], rows.at[1 - b],
                                 gsem)
            pltpu.make_async_copy(tbl.at[src_all.at[j]], rows.at[b],
                                  gsem).wait()
            pltpu.sync_copy(rows.at[b], s_sh.at[dst_all.at[j]], add=True)

            @pl.when((j < NCH // 2) == count_first_half)
            def _():
                pltpu.sync_copy(ones_b, d_sh.at[dst_all.at[j]], add=True)
            return _
        lax.fori_loop(0, NCH, _iter, None)

    @pl.when(cid == 0)
    def _():
        _run(nflo_hbm, True)

    @pl.when(cid == 1)
    def _():
        _run(nfhi_hbm, False)

    plsc.subcore_barrier()

    rs = sid * RPT
    pltpu.sync_copy(s_sh.at[pl.ds(rs, RPT)],
                    s_out.at[pl.ds(cid * N_PAD + rs, RPT)])
    pltpu.sync_copy(d_sh.at[pl.ds(rs, RPT)],
                    d_out.at[pl.ds(cid * N_PAD + rs, RPT)])


_sc_hagg = pl.kernel(
    _sc_hagg_body,
    out_type=[jax.ShapeDtypeStruct((2 * N_PAD, D_H), jnp.float32),
              jax.ShapeDtypeStruct((2 * N_PAD, 16), jnp.float32)],
    mesh=_SC_MESH,
    compiler_params=pltpu.CompilerParams(use_tc_tiling_on_sc=False),
    scratch_types=[
        pltpu.VMEM((NCH, CH), jnp.int32),
        pltpu.VMEM((NCH, CH), jnp.int32),
        pltpu.VMEM((2, CH, D_H), jnp.float32),
        pltpu.VMEM((CH, 16), jnp.float32),
        pltpu.VMEM((ZR, D_H), jnp.float32),
        pltpu.VMEM((RPT, 16), jnp.float32),
        pltpu.VMEM_SHARED((N_PAD, D_H), jnp.float32),
        pltpu.VMEM_SHARED((N_PAD, 16), jnp.float32),
        pltpu.SemaphoreType.DMA,
    ],
)


NW = NC * NS
EPW_F = N_EDGES // NW
NCH_F = EPW_F // CH


def _sc_fasm_body(src2_hbm, dst2_hbm, fni_hbm, fnj_hbm, ein_hbm, f_out,
                  src_all, dst_all, abuf, bbuf, cbuf, obuf, gsem):
    cid = lax.axis_index("c")
    sid = lax.axis_index("s")
    wid = sid * NC + cid
    base = wid * EPW_F

    pltpu.sync_copy(src2_hbm.at[pl.ds(wid * NCH_F, NCH_F)], src_all)
    pltpu.sync_copy(dst2_hbm.at[pl.ds(wid * NCH_F, NCH_F)], dst_all)

    def _start(j, b):
        pltpu.async_copy(fni_hbm.at[src_all.at[j]], abuf.at[b], gsem)
        pltpu.async_copy(fnj_hbm.at[dst_all.at[j]], bbuf.at[b], gsem)
        pltpu.async_copy(ein_hbm.at[pl.ds(base + j * CH, CH)], cbuf.at[b],
                         gsem)

    def _wait(j, b):
        pltpu.make_async_copy(fni_hbm.at[src_all.at[j]], abuf.at[b],
                              gsem).wait()
        pltpu.make_async_copy(fnj_hbm.at[dst_all.at[j]], bbuf.at[b],
                              gsem).wait()
        pltpu.make_async_copy(ein_hbm.at[pl.ds(base + j * CH, CH)],
                              cbuf.at[b], gsem).wait()

    _start(0, 0)

    def _iter(j, _):
        b = lax.rem(j, 2)

        @pl.when(j + 1 < NCH_F)
        def _():
            _start(j + 1, 1 - b)

        _wait(j, b)

        def _edge(i, _):
            x = abuf[b, i] + bbuf[b, i] + cbuf[b, i]
            obuf[b, i] = jnp.maximum(x, x * 0.01)
            return _
        lax.fori_loop(0, CH, _edge, None)

        pltpu.sync_copy(obuf.at[b], f_out.at[pl.ds(base + j * CH, CH)])
        return _
    lax.fori_loop(0, NCH_F, _iter, None)


_sc_fasm = pl.kernel(
    _sc_fasm_body,
    out_type=jax.ShapeDtypeStruct((N_EDGES, D_E), jnp.float32),
    mesh=_SC_MESH,
    compiler_params=pltpu.CompilerParams(use_tc_tiling_on_sc=False),
    scratch_types=[
        pltpu.VMEM((NCH_F, CH), jnp.int32),
        pltpu.VMEM((NCH_F, CH), jnp.int32),
        pltpu.VMEM((2, CH, D_E), jnp.float32),
        pltpu.VMEM((2, CH, D_E), jnp.float32),
        pltpu.VMEM((2, CH, D_E), jnp.float32),
        pltpu.VMEM((2, CH, D_E), jnp.float32),
        pltpu.SemaphoreType.DMA,
    ],
)


def _tc_node_tables_body(x_ref, wni_ref, wnj_ref, oni_ref, onj_ref):
    x = x_ref[...]
    dn = (((1,), (1,)), ((), ()))
    oni_ref[...] = lax.dot_general(x, wni_ref[...], dn,
                                   preferred_element_type=jnp.float32)
    onj_ref[...] = lax.dot_general(x, wnj_ref[...], dn,
                                   preferred_element_type=jnp.float32)


def _tc_node_tables(nfeats, W_ni, W_nj):
    blk = 1000
    return pl.pallas_call(
        _tc_node_tables_body,
        grid=(N_NODES // blk,),
        in_specs=[pl.BlockSpec((blk, D_IN), lambda i: (i, 0)),
                  pl.BlockSpec((D_E, D_IN), lambda i: (0, 0)),
                  pl.BlockSpec((D_E, D_IN), lambda i: (0, 0))],
        out_specs=[pl.BlockSpec((blk, D_E), lambda i: (i, 0)),
                   pl.BlockSpec((blk, D_E), lambda i: (i, 0))],
        out_shape=[jax.ShapeDtypeStruct((N_NODES, D_E), jnp.float32),
                   jax.ShapeDtypeStruct((N_NODES, D_E), jnp.float32)],
    )(nfeats, W_ni, W_nj)


def _tc_edge_proj_body(e_ref, w_ref, b_ref, o_ref):
    dn = (((1,), (1,)), ((), ()))
    o_ref[...] = lax.dot_general(e_ref[...], w_ref[...], dn,
                                 preferred_element_type=jnp.float32) + b_ref[...]


def _tc_edge_proj(efeats, W_fij, bias_e):
    blk = 2000
    return pl.pallas_call(
        _tc_edge_proj_body,
        grid=(N_EDGES // blk,),
        in_specs=[pl.BlockSpec((blk, D_E), lambda i: (i, 0)),
                  pl.BlockSpec((D_E, D_E), lambda i: (0, 0)),
                  pl.BlockSpec((1, D_E), lambda i: (0, 0))],
        out_specs=pl.BlockSpec((blk, D_E), lambda i: (i, 0)),
        out_shape=jax.ShapeDtypeStruct((N_EDGES, D_E), jnp.float32),
    )(efeats, W_fij, bias_e)


def _tc_node_out_body(s_ref, d_ref, w_ref, b_ref, o_ref):
    s = jnp.concatenate([s_ref[0], s_ref[1]], axis=1)
    deg = (d_ref[0] + d_ref[1])[:, 0:1]
    acc = lax.dot_general(s, w_ref[...], (((1,), (1,)), ((), ())),
                          preferred_element_type=jnp.float32)
    o_ref[...] = acc / jnp.maximum(deg, 1.0) + jnp.where(deg > 0.0,
                                                         b_ref[...], 0.0)


def _tc_node_out(s2, d2, W_node, b_node):
    blk = 1000
    return pl.pallas_call(
        _tc_node_out_body,
        grid=(N_NODES // blk,),
        in_specs=[pl.BlockSpec((2, blk, D_H), lambda i: (0, i, 0)),
                  pl.BlockSpec((2, blk, 16), lambda i: (0, i, 0)),
                  pl.BlockSpec((D_IN, D_IN), lambda i: (0, 0)),
                  pl.BlockSpec((1, D_IN), lambda i: (0, 0))],
        out_specs=pl.BlockSpec((blk, D_IN), lambda i: (i, 0)),
        out_shape=jax.ShapeDtypeStruct((N_NODES, D_IN), jnp.float32),
    )(s2, d2, W_node, b_node)


def kernel(nfeats, efeats, edge_index, W_node, b_node, W_ni, W_nj, W_fij, bias_e):
    src2 = edge_index[0].reshape(IDX_ROWS, CH)
    dst2 = edge_index[1].reshape(IDX_ROWS, CH)
    nf_lo = nfeats[:, :D_H]
    nf_hi = nfeats[:, D_H:]

    s2, d2 = _sc_hagg(src2, dst2, nf_lo, nf_hi)
    f_ni, f_nj = _tc_node_tables(nfeats, W_ni, W_nj)
    e_in = _tc_edge_proj(efeats, W_fij, bias_e.reshape(1, D_E))
    f_out = _sc_fasm(src2, dst2, f_ni, f_nj, e_in)
    h_out = _tc_node_out(s2.reshape(2, N_PAD, D_H),
                         d2.reshape(2, N_PAD, 16),
                         W_node, b_node.reshape(1, D_IN))
    return h_out, f_out


# speedup vs baseline: 6.5199x; 1.4402x over previous
import functools

import jax
import jax.numpy as jnp
from jax import lax
from jax.experimental import pallas as pl
from jax.experimental.pallas import tpu as pltpu
from jax.experimental.pallas import tpu_sc as plsc

N_NODES = 10000
N_EDGES = 320000
D_IN = 128
D_H = D_IN // 2
D_E = 16

NC = 2
NS = 16
CH = 80
IDX_ROWS = N_EDGES // CH

EPW = N_EDGES // NS
NCH = EPW // CH
N_PAD = 10240
RPT = N_PAD // NS
ZR = 128

_SC_MESH = plsc.VectorSubcoreMesh(core_axis_name="c", subcore_axis_name="s")


def _sc_hagg_body(src2_hbm, dst2_hbm, nflo_hbm, nfhi_hbm, s_out, d_out,
                  src_all, dst_all, rows, ones_b, zbuf, zd, s_sh, d_sh, gsem):
    cid = lax.axis_index("c")
    sid = lax.axis_index("s")

    z16 = jnp.zeros((16,), jnp.float32)
    one16 = jnp.ones((16,), jnp.float32)

    def _zrow(i, _):
        for c in range(D_H // 16):
            zbuf[i, pl.ds(c * 16, 16)] = z16
        return _
    lax.fori_loop(0, ZR, _zrow, None)

    def _zdeg(i, _):
        zd[i] = z16
        return _
    lax.fori_loop(0, RPT, _zdeg, None)

    def _ones(i, _):
        ones_b[i] = one16
        return _
    lax.fori_loop(0, CH, _ones, None)

    pltpu.sync_copy(src2_hbm.at[pl.ds(sid * NCH, NCH)], src_all)
    pltpu.sync_copy(dst2_hbm.at[pl.ds(sid * NCH, NCH)], dst_all)

    for k in range(RPT // ZR):
        pltpu.sync_copy(zbuf, s_sh.at[pl.ds(sid * RPT + k * ZR, ZR)])
    pltpu.sync_copy(zd, d_sh.at[pl.ds(sid * RPT, RPT)])
    plsc.subcore_barrier()

    def _run(tbl, count_first_half):
        pltpu.async_copy(tbl.at[src_all.at[0]], rows.at[0], gsem)

        def _iter(j, _):
            b = lax.rem(j, 2)
            @pl.when(j + 1 < NCH)
            def _():
                pltpu.async_copy(tbl.at[src_all.at[j + 1]], rows.at[1 - b],
                                 gsem)
            pltpu.make_async_copy(tbl.at[src_all.at[j]], rows.at[b],
                                  gsem).wait()
            pltpu.sync_copy(rows.at[b], s_sh.at[dst_all.at[j]], add=True)

            @pl.when((j < NCH // 2) == count_first_half)
            def _():
                pltpu.sync_copy(ones_b, d_sh.at[dst_all.at[j]], add=True)
            return _
        lax.fori_loop(0, NCH, _iter, None)

    @pl.when(cid == 0)
    def _():
        _run(nflo_hbm, True)

    @pl.when(cid == 1)
    def _():
        _run(nfhi_hbm, False)

    plsc.subcore_barrier()

    rs = sid * RPT
    pltpu.sync_copy(s_sh.at[pl.ds(rs, RPT)],
                    s_out.at[pl.ds(cid * N_PAD + rs, RPT)])
    pltpu.sync_copy(d_sh.at[pl.ds(rs, RPT)],
                    d_out.at[pl.ds(cid * N_PAD + rs, RPT)])


_sc_hagg = pl.kernel(
    _sc_hagg_body,
    out_type=[jax.ShapeDtypeStruct((2 * N_PAD, D_H), jnp.float32),
              jax.ShapeDtypeStruct((2 * N_PAD, 16), jnp.float32)],
    mesh=_SC_MESH,
    compiler_params=pltpu.CompilerParams(use_tc_tiling_on_sc=False),
    scratch_types=[
        pltpu.VMEM((NCH, CH), jnp.int32),
        pltpu.VMEM((NCH, CH), jnp.int32),
        pltpu.VMEM((2, CH, D_H), jnp.float32),
        pltpu.VMEM((CH, 16), jnp.float32),
        pltpu.VMEM((ZR, D_H), jnp.float32),
        pltpu.VMEM((RPT, 16), jnp.float32),
        pltpu.VMEM_SHARED((N_PAD, D_H), jnp.float32),
        pltpu.VMEM_SHARED((N_PAD, 16), jnp.float32),
        pltpu.SemaphoreType.DMA,
    ],
)


NW = NC * NS
EPW_F = N_EDGES // NW
NCH_F = EPW_F // CH


CR = CH * D_E // 128


def _sc_fasm_body(dep_hbm, src2_hbm, dst2_hbm, fni_hbm, fnj_hbm, ein_hbm,
                  f_out, src_all, dst_all, abuf, bbuf, cbuf, obuf, gsem):
    cid = lax.axis_index("c")
    sid = lax.axis_index("s")
    wid = sid * NC + cid
    base = wid * EPW_F
    rbase = base * D_E // 128

    pltpu.sync_copy(src2_hbm.at[pl.ds(wid * NCH_F, NCH_F)], src_all)
    pltpu.sync_copy(dst2_hbm.at[pl.ds(wid * NCH_F, NCH_F)], dst_all)

    def _start(j, b):
        pltpu.async_copy(fni_hbm.at[src_all.at[j]], abuf.at[b], gsem)
        pltpu.async_copy(fnj_hbm.at[dst_all.at[j]], bbuf.at[b], gsem)
        pltpu.async_copy(ein_hbm.at[pl.ds(rbase + j * CR, CR)], cbuf.at[b],
                         gsem)

    def _wait(j, b):
        pltpu.make_async_copy(fni_hbm.at[src_all.at[j]], abuf.at[b],
                              gsem).wait()
        pltpu.make_async_copy(fnj_hbm.at[dst_all.at[j]], bbuf.at[b],
                              gsem).wait()
        pltpu.make_async_copy(ein_hbm.at[pl.ds(rbase + j * CR, CR)],
                              cbuf.at[b], gsem).wait()

    _start(0, 0)

    def _iter(j, _):
        b = lax.rem(j, 2)

        @pl.when(j + 1 < NCH_F)
        def _():
            _start(j + 1, 1 - b)

        _wait(j, b)

        def _row(t, _):
            for r in range(8):
                i = t * 8 + r
                x = abuf[b, i] + bbuf[b, i] + cbuf[b, t, pl.ds(r * D_E, D_E)]
                obuf[b, i] = jnp.maximum(x, x * 0.01)
            return _
        lax.fori_loop(0, CR, _row, None)

        pltpu.sync_copy(obuf.at[b], f_out.at[pl.ds(base + j * CH, CH)])
        return _
    lax.fori_loop(0, NCH_F, _iter, None)


_sc_fasm = pl.kernel(
    _sc_fasm_body,
    out_type=jax.ShapeDtypeStruct((N_EDGES, D_E), jnp.float32),
    mesh=_SC_MESH,
    compiler_params=pltpu.CompilerParams(use_tc_tiling_on_sc=False),
    scratch_types=[
        pltpu.VMEM((NCH_F, CH), jnp.int32),
        pltpu.VMEM((NCH_F, CH), jnp.int32),
        pltpu.VMEM((2, CH, D_E), jnp.float32),
        pltpu.VMEM((2, CH, D_E), jnp.float32),
        pltpu.VMEM((2, CR, 128), jnp.float32),
        pltpu.VMEM((2, CH, D_E), jnp.float32),
        pltpu.SemaphoreType.DMA,
    ],
)


def _tc_node_tables_body(x_ref, wni_ref, wnj_ref, oni_ref, onj_ref):
    x = x_ref[...]
    dn = (((1,), (1,)), ((), ()))
    oni_ref[...] = lax.dot_general(x, wni_ref[...], dn,
                                   preferred_element_type=jnp.float32)
    onj_ref[...] = lax.dot_general(x, wnj_ref[...], dn,
                                   preferred_element_type=jnp.float32)


def _tc_node_tables(nfeats, W_ni, W_nj):
    blk = 1000
    return pl.pallas_call(
        _tc_node_tables_body,
        grid=(N_NODES // blk,),
        in_specs=[pl.BlockSpec((blk, D_IN), lambda i: (i, 0)),
                  pl.BlockSpec((D_E, D_IN), lambda i: (0, 0)),
                  pl.BlockSpec((D_E, D_IN), lambda i: (0, 0))],
        out_specs=[pl.BlockSpec((blk, D_E), lambda i: (i, 0)),
                   pl.BlockSpec((blk, D_E), lambda i: (i, 0))],
        out_shape=[jax.ShapeDtypeStruct((N_NODES, D_E), jnp.float32),
                   jax.ShapeDtypeStruct((N_NODES, D_E), jnp.float32)],
    )(nfeats, W_ni, W_nj)


E_ROWS = N_EDGES * D_E // 128


def _tc_edge_proj_body(e_ref, w_ref, b_ref, o_ref):
    dn = (((1,), (0,)), ((), ()))
    o_ref[...] = lax.dot_general(e_ref[...], w_ref[...], dn,
                                 preferred_element_type=jnp.float32) + b_ref[...]


def _tc_edge_proj(e128, bd, b128):
    blk = 2000
    return pl.pallas_call(
        _tc_edge_proj_body,
        grid=(E_ROWS // blk,),
        in_specs=[pl.BlockSpec((blk, 128), lambda i: (i, 0)),
                  pl.BlockSpec((128, 128), lambda i: (0, 0)),
                  pl.BlockSpec((1, 128), lambda i: (0, 0))],
        out_specs=pl.BlockSpec((blk, 128), lambda i: (i, 0)),
        out_shape=jax.ShapeDtypeStruct((E_ROWS, 128), jnp.float32),
    )(e128, bd, b128)


def _tc_node_out_body(s_ref, d_ref, w_ref, b_ref, o_ref):
    s = jnp.concatenate([s_ref[0], s_ref[1]], axis=1)
    deg = (d_ref[0] + d_ref[1])[:, 0:1]
    acc = lax.dot_general(s, w_ref[...], (((1,), (1,)), ((), ())),
                          preferred_element_type=jnp.float32)
    o_ref[...] = acc / jnp.maximum(deg, 1.0) + jnp.where(deg > 0.0,
                                                         b_ref[...], 0.0)


def _tc_node_out(s2, d2, W_node, b_node):
    blk = 1000
    return pl.pallas_call(
        _tc_node_out_body,
        grid=(N_NODES // blk,),
        in_specs=[pl.BlockSpec((2, blk, D_H), lambda i: (0, i, 0)),
                  pl.BlockSpec((2, blk, 16), lambda i: (0, i, 0)),
                  pl.BlockSpec((D_IN, D_IN), lambda i: (0, 0)),
                  pl.BlockSpec((1, D_IN), lambda i: (0, 0))],
        out_specs=pl.BlockSpec((blk, D_IN), lambda i: (i, 0)),
        out_shape=jax.ShapeDtypeStruct((N_NODES, D_IN), jnp.float32),
    )(s2, d2, W_node, b_node)


def kernel(nfeats, efeats, edge_index, W_node, b_node, W_ni, W_nj, W_fij, bias_e):
    src2 = edge_index[0].reshape(IDX_ROWS, CH)
    dst2 = edge_index[1].reshape(IDX_ROWS, CH)
    nf_lo = nfeats[:, :D_H]
    nf_hi = nfeats[:, D_H:]

    s2, d2 = _sc_hagg(src2, dst2, nf_lo, nf_hi)
    f_ni, f_nj = _tc_node_tables(nfeats, W_ni, W_nj)
    e128 = efeats.reshape(E_ROWS, 128)
    bd = jnp.kron(jnp.eye(8, dtype=jnp.float32), W_fij.T)
    b128 = jnp.tile(bias_e, 8).reshape(1, 128)
    e_in = _tc_edge_proj(e128, bd, b128)
    dep = s2[:8]
    f_out = _sc_fasm(dep, src2, dst2, f_ni, f_nj, e_in)
    h_out = _tc_node_out(s2.reshape(2, N_PAD, D_H),
                         d2.reshape(2, N_PAD, 16),
                         W_node, b_node.reshape(1, D_IN))
    return h_out, f_out
